# baseline (speedup 1.0000x reference)
import functools

import jax
import jax.numpy as jnp
from jax import lax
from jax.experimental import pallas as pl
from jax.experimental.pallas import tpu as pltpu
from jax.experimental.pallas import tpu_sc as plsc

D_MODEL = 64
NUM_CORES = 2
NUM_SUBCORES = 16
NW = NUM_CORES * NUM_SUBCORES
B = 4096
S = 200
CB = B // NW
L = 16
PHS = 4
NPH = S // PHS


def _emb_kernel(table_hbm, idx_hbm, out_hbm,
                idx_v, hi_a, hi_b, buf_a, buf_b,
                out_a, out_b,
                gsem_a, gsem_b, wsem_a, wsem_b):
    his = (hi_a, hi_b)
    bufs = (buf_a, buf_b)
    outs = (out_a, out_b)
    gsems = (gsem_a, gsem_b)
    wsems = (wsem_a, wsem_b)
    wid = lax.axis_index("c") * NUM_SUBCORES + lax.axis_index("s")
    b0 = wid * CB
    pltpu.sync_copy(idx_hbm.at[wid], idx_v)

    lanes = lax.iota(jnp.int32, L)

    def prep_hi(p, hi_ref):
        for si in range(PHS):
            for g in range(CB // L):
                xv = idx_v[p * PHS + si, pl.ds(g * L, L)]
                m = (xv >= HALF).astype(jnp.int32)
                hi_ref[pl.ds(si * CB + g * L, L)] = lax.shift_left(xv - m * HALF, 1) + m

    def fire(p, hi_ref, buf, sem):
        prep_hi(p, hi_ref)
        pltpu.async_copy(table_hbm.at[hi_ref], buf, sem)

    def wait_gather(p, hi_ref, buf, sem):
        pltpu.make_async_copy(table_hbm.at[hi_ref], buf, sem).wait()

    def select(si, buf, out_t):
        for g in range(CB // L):
            bids = lanes + (si * CB + g * L)

            @plsc.parallel_loop(0, D_MODEL, unroll=8)
            def _(d):
                vals = plsc.load_gather(buf, [bids, jnp.broadcast_to(d, (L,))])
                out_t[lax.div(d, 8), lax.rem(d, 8), pl.ds(g * L, L)] = vals

    def write(s, out_t, sem):
        pltpu.async_copy(out_t, out_hbm.at[s, :, wid], sem)

    def wait_write(s, out_t, sem):
        pltpu.make_async_copy(out_t, out_hbm.at[s, :, wid], sem).wait()

    def visit(p, k, do_fire, first):
        if do_fire:
            fire(p + 1, his[1 - k], bufs[1 - k], gsems[1 - k])
        wait_gather(p, his[k], bufs[k], gsems[k])
        for si in range(PHS):
            s = p * PHS + si
            if not (first and si < 2):
                wait_write(s - 2, outs[si % 2], wsems[si % 2])
            select(si, bufs[k], outs[si % 2])
            write(s, outs[si % 2], wsems[si % 2])

    fire(0, his[0], bufs[0], gsems[0])

    visit(0, 0, True, True)

    @pl.loop(0, (NPH - 2) // 2)
    def _(pp):
        visit(2 * pp + 1, 1, True, False)
        visit(2 * pp + 2, 0, True, False)

    visit(NPH - 1, 1, False, False)

    wait_write(S - 2, out_a, wsem_a)
    wait_write(S - 1, out_b, wsem_b)


TBLK = 4096
NTB = 123
HALF = NTB * TBLK


def _tpose_kernel(lo_ref, hi_ref, o_ref):
    o_ref[:, 0:64] = jnp.swapaxes(lo_ref[...], 0, 1)
    o_ref[:, 64:128] = jnp.swapaxes(hi_ref[...], 0, 1)


def _transpose_table(t_t):
    return pl.pallas_call(
        _tpose_kernel,
        grid=(NTB,),
        in_specs=[
            pl.BlockSpec((64, TBLK), lambda i: (0, i)),
            pl.BlockSpec(
                (64, TBLK),
                lambda i: (0, jnp.minimum(i + NTB, (1_000_000 + TBLK - 1) // TBLK - 1)),
            ),
        ],
        out_specs=pl.BlockSpec((TBLK, 128), lambda i: (i, 0)),
        out_shape=jax.ShapeDtypeStruct((HALF, 128), jnp.float32),
    )(t_t, t_t)


@jax.jit
def _embed(table128, idx3):
    run = functools.partial(
        pl.kernel,
        out_type=jax.ShapeDtypeStruct((S, 8, NW, 8, CB), jnp.float32),
        mesh=plsc.VectorSubcoreMesh(core_axis_name="c", subcore_axis_name="s"),
        scratch_types=(
            [pltpu.VMEM((S, CB), jnp.int32)]
            + [pltpu.VMEM((PHS * CB,), jnp.int32) for _ in range(2)]
            + [pltpu.VMEM((PHS * CB, D_MODEL), jnp.float32) for _ in range(2)]
            + [pltpu.VMEM((8, 8, CB), jnp.float32) for _ in range(2)]
            + [pltpu.SemaphoreType.DMA for _ in range(4)]
        ),
        compiler_params=pltpu.CompilerParams(needs_layout_passes=False, use_tc_tiling_on_sc=False),
    )(_emb_kernel)
    return run(table128, idx3)


def kernel(x, table):
    table64 = _transpose_table(table.T).reshape(2 * HALF, D_MODEL)
    idx3 = x.T.reshape(S, NW, CB).transpose(1, 0, 2).astype(jnp.int32)
    out5 = _embed(table64, idx3)
    out = out5.transpose(0, 1, 3, 2, 4).reshape(S, D_MODEL, B)
    return out.transpose(2, 0, 1)


# speedup vs baseline: 2.1057x; 2.1057x over previous
import functools

import jax
import jax.numpy as jnp
from jax import lax
from jax.experimental import pallas as pl
from jax.experimental.pallas import tpu as pltpu
from jax.experimental.pallas import tpu_sc as plsc

D_MODEL = 64
NUM_CORES = 2
NUM_SUBCORES = 16
NW = NUM_CORES * NUM_SUBCORES
B = 4096
S = 200
CB = B // NW
L = 16
PHS = 4
NPH = S // PHS


def _emb_kernel(table_hbm, idx_hbm, out_hbm,
                idx_v, hi_a, hi_b, buf_a, buf_b,
                out_a, out_b,
                gsem_a, gsem_b, wsem_a, wsem_b):
    his = (hi_a, hi_b)
    bufs = (buf_a, buf_b)
    outs = (out_a, out_b)
    gsems = (gsem_a, gsem_b)
    wsems = (wsem_a, wsem_b)
    wid = lax.axis_index("c") * NUM_SUBCORES + lax.axis_index("s")
    b0 = wid * CB
    pltpu.sync_copy(idx_hbm.at[wid], idx_v)

    lanes = lax.iota(jnp.int32, L)

    def prep_hi(p, hi_ref):
        for si in range(PHS):
            for g in range(CB // L):
                xv = idx_v[p * PHS + si, pl.ds(g * L, L)]
                m = (xv >= HALF).astype(jnp.int32)
                hi_ref[pl.ds(si * CB + g * L, L)] = lax.shift_left(xv - m * HALF, 1) + m

    def fire(p, hi_ref, buf, sem):
        prep_hi(p, hi_ref)
        pltpu.async_copy(table_hbm.at[hi_ref], buf, sem)

    def wait_gather(p, hi_ref, buf, sem):
        pltpu.make_async_copy(table_hbm.at[hi_ref], buf, sem).wait()

    def select(si, buf, out_t):
        for g in range(CB // L):
            bids = lanes + (si * CB + g * L)
            bl = lanes + (g * L)

            @plsc.parallel_loop(0, D_MODEL, unroll=8)
            def _(d):
                dd = jnp.bitwise_and(d + lanes, D_MODEL - 1)
                vals = plsc.load_gather(buf, [bids, dd])
                plsc.store_scatter(
                    out_t,
                    [lax.shift_right_logical(dd, 3), jnp.bitwise_and(dd, 7), bl],
                    vals,
                )

    def write(s, out_t, sem):
        pltpu.async_copy(out_t, out_hbm.at[s, :, wid], sem)

    def wait_write(s, out_t, sem):
        pltpu.make_async_copy(out_t, out_hbm.at[s, :, wid], sem).wait()

    def visit(p, k, do_fire, first):
        if do_fire:
            fire(p + 1, his[1 - k], bufs[1 - k], gsems[1 - k])
        wait_gather(p, his[k], bufs[k], gsems[k])
        for si in range(PHS):
            s = p * PHS + si
            if not (first and si < 2):
                wait_write(s - 2, outs[si % 2], wsems[si % 2])
            select(si, bufs[k], outs[si % 2])
            write(s, outs[si % 2], wsems[si % 2])

    fire(0, his[0], bufs[0], gsems[0])

    visit(0, 0, True, True)

    @pl.loop(0, (NPH - 2) // 2)
    def _(pp):
        visit(2 * pp + 1, 1, True, False)
        visit(2 * pp + 2, 0, True, False)

    visit(NPH - 1, 1, False, False)

    wait_write(S - 2, out_a, wsem_a)
    wait_write(S - 1, out_b, wsem_b)


TBLK = 4096
NTB = 123
HALF = NTB * TBLK


def _tpose_kernel(lo_ref, hi_ref, o_ref):
    o_ref[:, 0:64] = jnp.swapaxes(lo_ref[...], 0, 1)
    o_ref[:, 64:128] = jnp.swapaxes(hi_ref[...], 0, 1)


def _transpose_table(t_t):
    return pl.pallas_call(
        _tpose_kernel,
        grid=(NTB,),
        in_specs=[
            pl.BlockSpec((64, TBLK), lambda i: (0, i)),
            pl.BlockSpec(
                (64, TBLK),
                lambda i: (0, jnp.minimum(i + NTB, (1_000_000 + TBLK - 1) // TBLK - 1)),
            ),
        ],
        out_specs=pl.BlockSpec((TBLK, 128), lambda i: (i, 0)),
        out_shape=jax.ShapeDtypeStruct((HALF, 128), jnp.float32),
    )(t_t, t_t)


@jax.jit
def _embed(table128, idx3):
    run = functools.partial(
        pl.kernel,
        out_type=jax.ShapeDtypeStruct((S, 8, NW, 8, CB), jnp.float32),
        mesh=plsc.VectorSubcoreMesh(core_axis_name="c", subcore_axis_name="s"),
        scratch_types=(
            [pltpu.VMEM((S, CB), jnp.int32)]
            + [pltpu.VMEM((PHS * CB,), jnp.int32) for _ in range(2)]
            + [pltpu.VMEM((PHS * CB, D_MODEL), jnp.float32) for _ in range(2)]
            + [pltpu.VMEM((8, 8, CB), jnp.float32) for _ in range(2)]
            + [pltpu.SemaphoreType.DMA for _ in range(4)]
        ),
        compiler_params=pltpu.CompilerParams(needs_layout_passes=False, use_tc_tiling_on_sc=False),
    )(_emb_kernel)
    return run(table128, idx3)


def kernel(x, table):
    table64 = _transpose_table(table.T).reshape(2 * HALF, D_MODEL)
    idx3 = x.T.reshape(S, NW, CB).transpose(1, 0, 2).astype(jnp.int32)
    out5 = _embed(table64, idx3)
    out = out5.transpose(0, 1, 3, 2, 4).reshape(S, D_MODEL, B)
    return out.transpose(2, 0, 1)


# speedup vs baseline: 2.2679x; 1.0771x over previous
import functools

import jax
import jax.numpy as jnp
from jax import lax
from jax.experimental import pallas as pl
from jax.experimental.pallas import tpu as pltpu
from jax.experimental.pallas import tpu_sc as plsc

D_MODEL = 64
NUM_CORES = 2
NUM_SUBCORES = 16
NW = NUM_CORES * NUM_SUBCORES
B = 4096
S = 200
CB = B // NW
L = 16
PHS = 4
NPH = S // PHS


def _emb_kernel(table_hbm, idx_hbm, out_hbm,
                idx_v, hi_a, hi_b, buf_a, buf_b,
                out_a, out_b,
                gsem_a, gsem_b, wsem_a, wsem_b):
    his = (hi_a, hi_b)
    bufs = (buf_a, buf_b)
    outs = (out_a, out_b)
    gsems = (gsem_a, gsem_b)
    wsems = (wsem_a, wsem_b)
    wid = lax.axis_index("c") * NUM_SUBCORES + lax.axis_index("s")
    b0 = wid * CB
    pltpu.sync_copy(idx_hbm.at[wid], idx_v)

    lanes = lax.iota(jnp.int32, L)

    def prep_hi(p, hi_ref):
        for si in range(PHS):
            for g in range(CB // L):
                xv = idx_v[p * PHS + si, pl.ds(g * L, L)]
                m = (xv >= HALF).astype(jnp.int32)
                hi_ref[pl.ds(si * CB + g * L, L)] = lax.shift_left(xv - m * HALF, 1) + m

    def fire(p, hi_ref, buf, sem):
        prep_hi(p, hi_ref)
        pltpu.async_copy(table_hbm.at[hi_ref], buf, sem)

    def wait_gather(p, hi_ref, buf, sem):
        pltpu.make_async_copy(table_hbm.at[hi_ref], buf, sem).wait()

    def select(si, buf, out_t):
        for g in range(CB // L):
            bids = lanes + (si * CB + g * L)
            bl = lanes + (g * L)

            @plsc.parallel_loop(0, D_MODEL, unroll=8)
            def _(d):
                dd = jnp.bitwise_and(d + lanes, D_MODEL - 1)
                vals = plsc.load_gather(buf, [bids, dd])
                plsc.store_scatter(
                    out_t,
                    [lax.shift_right_logical(dd, 3), jnp.bitwise_and(dd, 7), bl],
                    vals,
                )

    def write(s, out_t, sem):
        pltpu.async_copy(out_t, out_hbm.at[s, :, wid], sem)

    def wait_write(s, out_t, sem):
        pltpu.make_async_copy(out_t, out_hbm.at[s, :, wid], sem).wait()

    def visit(p, k, do_fire, first):
        if do_fire:
            fire(p + 1, his[1 - k], bufs[1 - k], gsems[1 - k])
        wait_gather(p, his[k], bufs[k], gsems[k])
        for si in range(PHS):
            s = p * PHS + si
            if not (first and si < 2):
                wait_write(s - 2, outs[si % 2], wsems[si % 2])
            select(si, bufs[k], outs[si % 2])
            write(s, outs[si % 2], wsems[si % 2])

    fire(0, his[0], bufs[0], gsems[0])

    visit(0, 0, True, True)

    @pl.loop(0, (NPH - 2) // 2)
    def _(pp):
        visit(2 * pp + 1, 1, True, False)
        visit(2 * pp + 2, 0, True, False)

    visit(NPH - 1, 1, False, False)

    wait_write(S - 2, out_a, wsem_a)
    wait_write(S - 1, out_b, wsem_b)


TBLK = 8192
NTB = 62
HALF = NTB * TBLK


def _tpose_kernel(lo_ref, hi_ref, o_ref):
    o_ref[:, 0:64] = jnp.swapaxes(lo_ref[...], 0, 1)
    o_ref[:, 64:128] = jnp.swapaxes(hi_ref[...], 0, 1)


def _transpose_table(t_t):
    return pl.pallas_call(
        _tpose_kernel,
        grid=(NTB,),
        in_specs=[
            pl.BlockSpec((64, TBLK), lambda i: (0, i)),
            pl.BlockSpec(
                (64, TBLK),
                lambda i: (0, jnp.minimum(i + NTB, (1_000_000 + TBLK - 1) // TBLK - 1)),
            ),
        ],
        out_specs=pl.BlockSpec((TBLK, 128), lambda i: (i, 0)),
        out_shape=jax.ShapeDtypeStruct((HALF, 128), jnp.float32),
    )(t_t, t_t)


@jax.jit
def _embed(table128, idx3):
    run = functools.partial(
        pl.kernel,
        out_type=jax.ShapeDtypeStruct((S, 8, NW, 8, CB), jnp.float32),
        mesh=plsc.VectorSubcoreMesh(core_axis_name="c", subcore_axis_name="s"),
        scratch_types=(
            [pltpu.VMEM((S, CB), jnp.int32)]
            + [pltpu.VMEM((PHS * CB,), jnp.int32) for _ in range(2)]
            + [pltpu.VMEM((PHS * CB, D_MODEL), jnp.float32) for _ in range(2)]
            + [pltpu.VMEM((8, 8, CB), jnp.float32) for _ in range(2)]
            + [pltpu.SemaphoreType.DMA for _ in range(4)]
        ),
        compiler_params=pltpu.CompilerParams(needs_layout_passes=False, use_tc_tiling_on_sc=False),
    )(_emb_kernel)
    return run(table128, idx3)


def kernel(x, table):
    table64 = _transpose_table(table.T).reshape(2 * HALF, D_MODEL)
    idx3 = x.T.reshape(S, NW, CB).transpose(1, 0, 2).astype(jnp.int32)
    out5 = _embed(table64, idx3)
    out = out5.transpose(0, 1, 3, 2, 4).reshape(S, D_MODEL, B)
    return out.transpose(2, 0, 1)


# speedup vs baseline: 2.3463x; 1.0345x over previous
import functools

import jax
import jax.numpy as jnp
from jax import lax
from jax.experimental import pallas as pl
from jax.experimental.pallas import tpu as pltpu
from jax.experimental.pallas import tpu_sc as plsc

D_MODEL = 64
NUM_CORES = 2
NUM_SUBCORES = 16
NW = NUM_CORES * NUM_SUBCORES
B = 4096
S = 200
CB = B // NW
L = 16
PHS = 4
NPH = S // PHS


def _emb_kernel(table_hbm, idx_hbm, out_hbm,
                idx_v, hi_a, hi_b, buf_a, buf_b,
                out_a, out_b,
                gsem_a, gsem_b, wsem_a, wsem_b):
    his = (hi_a, hi_b)
    bufs = (buf_a, buf_b)
    outs = (out_a, out_b)
    gsems = (gsem_a, gsem_b)
    wsems = (wsem_a, wsem_b)
    wid = lax.axis_index("c") * NUM_SUBCORES + lax.axis_index("s")
    b0 = wid * CB
    pltpu.sync_copy(idx_hbm.at[wid], idx_v)

    lanes = lax.iota(jnp.int32, L)

    def prep_hi(p, hi_ref):
        for si in range(PHS):
            for g in range(CB // L):
                xv = idx_v[p * PHS + si, pl.ds(g * L, L)]
                m = (xv >= HALF).astype(jnp.int32)
                hi_ref[pl.ds(si * CB + g * L, L)] = lax.shift_left(xv - m * HALF, 1) + m

    def fire(p, hi_ref, buf, sem):
        prep_hi(p, hi_ref)
        pltpu.async_copy(table_hbm.at[hi_ref], buf, sem)

    def wait_gather(p, hi_ref, buf, sem):
        pltpu.make_async_copy(table_hbm.at[hi_ref], buf, sem).wait()

    def select(si, buf, out_t):
        for g in range(CB // L):
            bids = lanes + (si * CB + g * L)
            bl = lanes + (g * L)

            @plsc.parallel_loop(0, D_MODEL, unroll=8)
            def _(d):
                dd = jnp.bitwise_and(d + lanes, D_MODEL - 1)
                vals = plsc.load_gather(buf, [bids, dd])
                plsc.store_scatter(
                    out_t,
                    [lax.shift_right_logical(dd, 3), jnp.bitwise_and(dd, 7), bl],
                    vals,
                )

    def write(s, out_t, sem):
        pltpu.async_copy(out_t, out_hbm.at[s, :, wid], sem)

    def wait_write(s, out_t, sem):
        pltpu.make_async_copy(out_t, out_hbm.at[s, :, wid], sem).wait()

    def visit(p, k, do_fire, first):
        if do_fire:
            fire(p + 1, his[1 - k], bufs[1 - k], gsems[1 - k])
        wait_gather(p, his[k], bufs[k], gsems[k])
        for si in range(PHS):
            s = p * PHS + si
            if not (first and si < 2):
                wait_write(s - 2, outs[si % 2], wsems[si % 2])
            select(si, bufs[k], outs[si % 2])
            write(s, outs[si % 2], wsems[si % 2])

    fire(0, his[0], bufs[0], gsems[0])

    visit(0, 0, True, True)

    @pl.loop(0, (NPH - 2) // 2)
    def _(pp):
        visit(2 * pp + 1, 1, True, False)
        visit(2 * pp + 2, 0, True, False)

    visit(NPH - 1, 1, False, False)

    wait_write(S - 2, out_a, wsem_a)
    wait_write(S - 1, out_b, wsem_b)


TBLK = 16384
NTB = 31
HALF = NTB * TBLK


def _tpose_kernel(lo_ref, hi_ref, o_ref):
    o_ref[:, 0:64] = jnp.swapaxes(lo_ref[...], 0, 1)
    o_ref[:, 64:128] = jnp.swapaxes(hi_ref[...], 0, 1)


def _transpose_table(t_t):
    return pl.pallas_call(
        _tpose_kernel,
        grid=(NTB,),
        in_specs=[
            pl.BlockSpec((64, TBLK), lambda i: (0, i)),
            pl.BlockSpec(
                (64, TBLK),
                lambda i: (0, jnp.minimum(i + NTB, (1_000_000 + TBLK - 1) // TBLK - 1)),
            ),
        ],
        out_specs=pl.BlockSpec((TBLK, 128), lambda i: (i, 0)),
        out_shape=jax.ShapeDtypeStruct((HALF, 128), jnp.float32),
    )(t_t, t_t)


@jax.jit
def _embed(table128, idx3):
    run = functools.partial(
        pl.kernel,
        out_type=jax.ShapeDtypeStruct((S, 8, NW, 8, CB), jnp.float32),
        mesh=plsc.VectorSubcoreMesh(core_axis_name="c", subcore_axis_name="s"),
        scratch_types=(
            [pltpu.VMEM((S, CB), jnp.int32)]
            + [pltpu.VMEM((PHS * CB,), jnp.int32) for _ in range(2)]
            + [pltpu.VMEM((PHS * CB, D_MODEL), jnp.float32) for _ in range(2)]
            + [pltpu.VMEM((8, 8, CB), jnp.float32) for _ in range(2)]
            + [pltpu.SemaphoreType.DMA for _ in range(4)]
        ),
        compiler_params=pltpu.CompilerParams(needs_layout_passes=False, use_tc_tiling_on_sc=False),
    )(_emb_kernel)
    return run(table128, idx3)


def kernel(x, table):
    table64 = _transpose_table(table.T).reshape(2 * HALF, D_MODEL)
    idx3 = x.T.reshape(S, NW, CB).transpose(1, 0, 2).astype(jnp.int32)
    out5 = _embed(table64, idx3)
    out = out5.transpose(0, 1, 3, 2, 4).reshape(S, D_MODEL, B)
    return out.transpose(2, 0, 1)


# speedup vs baseline: 2.3537x; 1.0032x over previous
import functools

import jax
import jax.numpy as jnp
from jax import lax
from jax.experimental import pallas as pl
from jax.experimental.pallas import tpu as pltpu
from jax.experimental.pallas import tpu_sc as plsc

D_MODEL = 64
NUM_CORES = 2
NUM_SUBCORES = 16
NW = NUM_CORES * NUM_SUBCORES
B = 4096
S = 200
CB = B // NW
L = 16
PHS = 4
NPH = S // PHS


def _emb_kernel(table_hbm, idx_hbm, out_hbm,
                idx_v, hi_a, hi_b, buf_a, buf_b,
                out_a, out_b,
                gsem_a, gsem_b, wsem_a, wsem_b):
    his = (hi_a, hi_b)
    bufs = (buf_a, buf_b)
    outs = (out_a, out_b)
    gsems = (gsem_a, gsem_b)
    wsems = (wsem_a, wsem_b)
    wid = lax.axis_index("c") * NUM_SUBCORES + lax.axis_index("s")
    pltpu.sync_copy(idx_hbm.at[wid], idx_v)

    lanes = lax.iota(jnp.int32, L)

    def prep_hi(p, hi_ref):
        for si in range(PHS):
            for g in range(CB // L):
                xv = idx_v[p * PHS + si, pl.ds(g * L, L)]
                m = (xv >= HALF).astype(jnp.int32)
                hi_ref[pl.ds(si * CB + g * L, L)] = lax.shift_left(xv - m * HALF, 1) + m

    def fire(p, hi_ref, buf, sem):
        prep_hi(p, hi_ref)
        pltpu.async_copy(table_hbm.at[hi_ref], buf, sem)

    def wait_gather(p, hi_ref, buf, sem):
        pltpu.make_async_copy(table_hbm.at[hi_ref], buf, sem).wait()

    def select(si, buf, out_t):
        for g in range(CB // L):
            bids = lanes + (si * CB + g * L)
            bl = lanes + (g * L)

            @plsc.parallel_loop(0, D_MODEL, unroll=8)
            def _(d):
                dd = jnp.bitwise_and(d + lanes, D_MODEL - 1)
                vals = plsc.load_gather(buf, [bids, dd])
                plsc.store_scatter(
                    out_t,
                    [lax.shift_right_logical(dd, 3), jnp.bitwise_and(dd, 7), bl],
                    vals,
                )

    def write(s, out_t, sem):
        pltpu.async_copy(out_t, out_hbm.at[s, :, wid], sem)

    def wait_write(s, out_t, sem):
        pltpu.make_async_copy(out_t, out_hbm.at[s, :, wid], sem).wait()

    def visit(p, k, do_fire, first):
        if do_fire:
            fire(p + 1, his[1 - k], bufs[1 - k], gsems[1 - k])
        wait_gather(p, his[k], bufs[k], gsems[k])
        for si in range(PHS):
            s = p * PHS + si
            if not (first and si < 2):
                wait_write(s - 2, outs[si % 2], wsems[si % 2])
            select(si, bufs[k], outs[si % 2])
            write(s, outs[si % 2], wsems[si % 2])

    fire(0, his[0], bufs[0], gsems[0])

    visit(0, 0, True, True)

    @pl.loop(0, (NPH - 2) // 2)
    def _(pp):
        visit(2 * pp + 1, 1, True, False)
        visit(2 * pp + 2, 0, True, False)

    visit(NPH - 1, 1, False, False)

    wait_write(S - 2, out_a, wsem_a)
    wait_write(S - 1, out_b, wsem_b)


TBLK = 16384
NTB = 31
HALF = NTB * TBLK


def _tpose_kernel(lo_ref, hi_ref, o_ref):
    o_ref[:, 0:64] = jnp.swapaxes(lo_ref[...], 0, 1)
    o_ref[:, 64:128] = jnp.swapaxes(hi_ref[...], 0, 1)


def _transpose_table(t_t):
    return pl.pallas_call(
        _tpose_kernel,
        grid=(NTB,),
        in_specs=[
            pl.BlockSpec((64, TBLK), lambda i: (0, i)),
            pl.BlockSpec(
                (64, TBLK),
                lambda i: (0, jnp.minimum(i + NTB, (1_000_000 + TBLK - 1) // TBLK - 1)),
            ),
        ],
        out_specs=pl.BlockSpec((TBLK, 128), lambda i: (i, 0)),
        out_shape=jax.ShapeDtypeStruct((HALF, 128), jnp.float32),
    )(t_t, t_t)


@jax.jit
def _embed(table128, idx3):
    run = functools.partial(
        pl.kernel,
        out_type=jax.ShapeDtypeStruct((S, 8, NW, 8, CB), jnp.float32),
        mesh=plsc.VectorSubcoreMesh(core_axis_name="c", subcore_axis_name="s"),
        scratch_types=(
            [pltpu.VMEM((S, CB), jnp.int32)]
            + [pltpu.VMEM((PHS * CB,), jnp.int32) for _ in range(2)]
            + [pltpu.VMEM((PHS * CB, D_MODEL), jnp.float32) for _ in range(2)]
            + [pltpu.VMEM((8, 8, CB), jnp.float32) for _ in range(2)]
            + [pltpu.SemaphoreType.DMA for _ in range(4)]
        ),
        compiler_params=pltpu.CompilerParams(needs_layout_passes=False, use_tc_tiling_on_sc=False),
    )(_emb_kernel)
    return run(table128, idx3)


def kernel(x, table):
    table64 = _transpose_table(table.T).reshape(2 * HALF, D_MODEL)
    idx3 = x.T.reshape(S, NW, CB).transpose(1, 0, 2).astype(jnp.int32)
    out5 = _embed(table64, idx3)
    out = out5.transpose(0, 1, 3, 2, 4).reshape(S, D_MODEL, B)
    return out.transpose(2, 0, 1)
